# baseline (speedup 1.0000x reference)
import functools

import jax
import jax.numpy as jnp
from jax import lax
from jax.experimental import pallas as pl
from jax.experimental.pallas import tpu as pltpu
from jax.experimental.pallas import tpu_sc as plsc

N_CLS = 1000
SEQ = 77
D = 512
N_CTX = 16
NA = 56
NA1 = 32
NA2 = NA - NA1
NB = 8
NTAIL = 5
NC, NS = 2, 16
NW = NC * NS
CPW = 32
LANES = 16


def _copy_row(src_ref, src_row, dst_ref, dst_row):
    for k in range(D // LANES):
        dst_ref[dst_row, pl.ds(k * LANES, LANES)] = (
            src_ref[src_row, pl.ds(k * LANES, LANES)]
        )


def _make_sc_call():
    mesh = plsc.VectorSubcoreMesh(
        core_axis_name="c", subcore_axis_name="s", num_cores=NC, num_subcores=NS
    )

    @functools.partial(
        pl.kernel,
        mesh=mesh,
        out_type=jax.ShapeDtypeStruct((N_CLS, SEQ, D), jnp.float32),
        scratch_types=[
            pltpu.VMEM((CPW, 1, NA1), jnp.int32),
            pltpu.VMEM((CPW, 1, NA2), jnp.int32),
            pltpu.VMEM((CPW, 1, NB), jnp.int32),
            pltpu.VMEM((8, D), jnp.float32),
            pltpu.VMEM((SEQ, D), jnp.float32),
            pltpu.VMEM((SEQ, D), jnp.float32),
            pltpu.VMEM((NB, D), jnp.float32),
            pltpu.VMEM((NB, D), jnp.float32),
            pltpu.SemaphoreType.DMA,
            pltpu.SemaphoreType.DMA,
            pltpu.SemaphoreType.DMA,
            pltpu.SemaphoreType.DMA,
        ],
    )
    def sc_kernel(idxa1_hbm, idxa2_hbm, idxb_hbm, table_hbm, cshift_hbm,
                  out_hbm, idxa1_v, idxa2_v, idxb_v, c15_v, blk0, blk1,
                  tl0, tl1, sga0, sga1, sgb0, sgb1):
        wid = lax.axis_index("s") * NC + lax.axis_index("c")
        blk = (blk0, blk1)
        tl = (tl0, tl1)
        sga = (sga0, sga1)
        sgb = (sgb0, sgb1)

        pltpu.sync_copy(idxa1_hbm.at[wid], idxa1_v)
        pltpu.sync_copy(idxa2_hbm.at[wid], idxa2_v)
        pltpu.sync_copy(idxb_hbm.at[wid], idxb_v)
        pltpu.sync_copy(cshift_hbm, blk0.at[pl.ds(0, 24)])
        pltpu.sync_copy(cshift_hbm, blk1.at[pl.ds(0, 24)])
        pltpu.sync_copy(cshift_hbm.at[pl.ds(16, 8)], c15_v)

        def gathers(c, b):
            return (
                pltpu.make_async_copy(
                    table_hbm.at[idxa1_v.at[c, 0]],
                    blk[b].at[pl.ds(N_CTX, NA1)], sga[b],
                ),
                pltpu.make_async_copy(
                    table_hbm.at[idxa2_v.at[c, 0]],
                    blk[b].at[pl.ds(N_CTX + NA1, NA2)], sga[b],
                ),
                pltpu.make_async_copy(table_hbm.at[idxb_v.at[c, 0]], tl[b], sgb[b]),
            )

        def issue_gathers(c, b):
            for cp in gathers(c, b):
                cp.start()

        def step(cc, b):
            g = wid * CPW + cc

            @pl.when(g < N_CLS)
            def _():
                @pl.when((cc + 1 < CPW) & (g + 1 < N_CLS))
                def _():
                    issue_gathers(cc + 1, 1 - b)

                for cp in gathers(cc, b):
                    cp.wait()
                _copy_row(blk[b], N_CTX, blk[b], 0)
                _copy_row(c15_v, 0, blk[b], N_CTX)
                for i in range(NTAIL):
                    _copy_row(tl[b], NB - NTAIL + i, blk[b], N_CTX + NA + i)
                pltpu.sync_copy(blk[b], out_hbm.at[g])

        issue_gathers(0, 0)

        def body(j, carry):
            step(2 * j, 0)
            step(2 * j + 1, 1)
            return carry

        lax.fori_loop(0, CPW // 2, body, 0)

    return sc_kernel


_sc_call = _make_sc_call()


def kernel(tokenized_prompts, token_embedding, ctx):
    tok = tokenized_prompts.astype(jnp.int32)
    gidxa = jnp.concatenate([tok[:, :1], tok[:, 1 + N_CTX:1 + N_CTX + NA - 1]],
                            axis=1)
    gidxb = tok[:, SEQ - NB:]
    gidxa = jnp.pad(gidxa, ((0, NW * CPW - N_CLS), (0, 0)))
    gidxb = jnp.pad(gidxb, ((0, NW * CPW - N_CLS), (0, 0)))
    gidxa1 = gidxa[:, :NA1].reshape(NW, CPW, 1, NA1)
    gidxa2 = gidxa[:, NA1:].reshape(NW, CPW, 1, NA2)
    gidxb = gidxb.reshape(NW, CPW, 1, NB)
    cshift = jnp.pad(ctx, ((1, 7), (0, 0)))
    return _sc_call(gidxa1, gidxa2, gidxb, token_embedding, cshift)


# speedup vs baseline: 1.0063x; 1.0063x over previous
import functools

import jax
import jax.numpy as jnp
from jax import lax
from jax.experimental import pallas as pl
from jax.experimental.pallas import tpu as pltpu
from jax.experimental.pallas import tpu_sc as plsc

N_CLS = 1000
SEQ = 77
D = 512
N_CTX = 16
NA = 56
NB = 8
NTAIL = 5
NC, NS = 2, 16
NW = NC * NS
CPW = 32
LANES = 16


def _copy_row(src_ref, src_row, dst_ref, dst_row):
    for k in range(D // LANES):
        dst_ref[dst_row, pl.ds(k * LANES, LANES)] = (
            src_ref[src_row, pl.ds(k * LANES, LANES)]
        )


def _make_sc_call():
    mesh = plsc.VectorSubcoreMesh(
        core_axis_name="c", subcore_axis_name="s", num_cores=NC, num_subcores=NS
    )

    @functools.partial(
        pl.kernel,
        mesh=mesh,
        out_type=jax.ShapeDtypeStruct((N_CLS, SEQ, D), jnp.float32),
        scratch_types=[
            pltpu.VMEM((CPW, 1, NA), jnp.int32),
            pltpu.VMEM((CPW, 1, NB), jnp.int32),
            pltpu.VMEM((8, D), jnp.float32),
            pltpu.VMEM((SEQ, D), jnp.float32),
            pltpu.VMEM((SEQ, D), jnp.float32),
            pltpu.VMEM((NB, D), jnp.float32),
            pltpu.VMEM((NB, D), jnp.float32),
            pltpu.SemaphoreType.DMA,
            pltpu.SemaphoreType.DMA,
            pltpu.SemaphoreType.DMA,
            pltpu.SemaphoreType.DMA,
        ],
    )
    def sc_kernel(idxa_hbm, idxb_hbm, table_hbm, cshift_hbm, out_hbm,
                  idxa_v, idxb_v, c15_v, blk0, blk1, tl0, tl1,
                  sga0, sga1, sgb0, sgb1):
        wid = lax.axis_index("s") * NC + lax.axis_index("c")
        blk = (blk0, blk1)
        tl = (tl0, tl1)
        sga = (sga0, sga1)
        sgb = (sgb0, sgb1)

        pltpu.sync_copy(idxa_hbm.at[wid], idxa_v)
        pltpu.sync_copy(idxb_hbm.at[wid], idxb_v)
        pltpu.sync_copy(cshift_hbm, blk0.at[pl.ds(0, 24)])
        pltpu.sync_copy(cshift_hbm, blk1.at[pl.ds(0, 24)])
        pltpu.sync_copy(cshift_hbm.at[pl.ds(16, 8)], c15_v)

        def gathers(c, b):
            return (
                pltpu.make_async_copy(
                    table_hbm.at[idxa_v.at[c, 0]],
                    blk[b].at[pl.ds(N_CTX, NA)], sga[b],
                ),
                pltpu.make_async_copy(table_hbm.at[idxb_v.at[c, 0]], tl[b], sgb[b]),
            )

        def issue_gathers(c, b):
            ga, gb = gathers(c, b)
            ga.start()
            gb.start()

        def step(cc, b):
            g = wid * CPW + cc

            @pl.when(g < N_CLS)
            def _():
                @pl.when((cc + 1 < CPW) & (g + 1 < N_CLS))
                def _():
                    issue_gathers(cc + 1, 1 - b)

                ga, gb = gathers(cc, b)
                ga.wait()
                gb.wait()
                _copy_row(blk[b], N_CTX, blk[b], 0)
                _copy_row(c15_v, 0, blk[b], N_CTX)
                for i in range(NTAIL):
                    _copy_row(tl[b], NB - NTAIL + i, blk[b], N_CTX + NA + i)
                pltpu.sync_copy(blk[b], out_hbm.at[g])

        issue_gathers(0, 0)

        def body(j, carry):
            step(2 * j, 0)
            step(2 * j + 1, 1)
            return carry

        lax.fori_loop(0, CPW // 2, body, 0)

    return sc_kernel


_sc_call = _make_sc_call()


def kernel(tokenized_prompts, token_embedding, ctx):
    tok = tokenized_prompts.astype(jnp.int32)
    gidxa = jnp.concatenate([tok[:, :1], tok[:, 1 + N_CTX:1 + N_CTX + NA - 1]],
                            axis=1)
    gidxb = tok[:, SEQ - NB:]
    gidxa = jnp.pad(gidxa, ((0, NW * CPW - N_CLS), (0, 0)))
    gidxb = jnp.pad(gidxb, ((0, NW * CPW - N_CLS), (0, 0)))
    gidxa = gidxa.reshape(NW, CPW, 1, NA)
    gidxb = gidxb.reshape(NW, CPW, 1, NB)
    cshift = jnp.pad(ctx, ((1, 7), (0, 0)))
    return _sc_call(gidxa, gidxb, token_embedding, cshift)


# speedup vs baseline: 1.8100x; 1.7986x over previous
import functools

import jax
import jax.numpy as jnp
from jax import lax
from jax.experimental import pallas as pl
from jax.experimental.pallas import tpu as pltpu
from jax.experimental.pallas import tpu_sc as plsc

N_CLS = 1000
SEQ = 77
D = 512
N_CTX = 16
NGPOS = SEQ - N_CTX
NC, NS = 2, 16
NW = NC * NS
CPW = 32
NCTX_CHUNKS = N_CTX * 4


def _make_sc_call():
    mesh = plsc.VectorSubcoreMesh(
        core_axis_name="c", subcore_axis_name="s", num_cores=NC, num_subcores=NS
    )

    @functools.partial(
        pl.kernel,
        mesh=mesh,
        out_type=jax.ShapeDtypeStruct((SEQ, N_CLS, D), jnp.float32),
        scratch_types=[
            pltpu.VMEM((SEQ, 1, CPW), jnp.int32),
            pltpu.VMEM((8 * N_CTX, D), jnp.float32),
            pltpu.VMEM((CPW, D), jnp.float32),
            pltpu.VMEM((CPW, D), jnp.float32),
            pltpu.SemaphoreType.DMA,
            pltpu.SemaphoreType.DMA,
        ],
    )
    def sc_kernel(idx_hbm, table_hbm, cbig_hbm, out_hbm,
                  idx_v, cb_v, gb0, gb1, sg0, sg1):
        wid = lax.axis_index("s") * NC + lax.axis_index("c")
        gb = (gb0, gb1)
        sg = (sg0, sg1)
        base = wid * CPW

        pltpu.sync_copy(idx_hbm.at[wid], idx_v)
        pltpu.sync_copy(cbig_hbm, cb_v)

        def gpos(i):
            return jnp.where(i == 0, 0, N_CTX + i)

        def gather(i, b):
            return pltpu.make_async_copy(
                table_hbm.at[idx_v.at[gpos(i), 0]], gb[b], sg[b]
            )

        def write_gather(i, b):
            r = gpos(i)

            @pl.when(base + CPW <= N_CLS)
            def _():
                pltpu.sync_copy(gb[b], out_hbm.at[r].at[pl.ds(base, CPW)])

            @pl.when(base + CPW > N_CLS)
            def _():
                pltpu.sync_copy(
                    gb[b].at[pl.ds(0, 8)],
                    out_hbm.at[r].at[pl.ds(base, 8)],
                )

        def write_ctx_chunk(q):
            j = q // 4
            off8 = (q % 4) * 8

            @pl.when(base + off8 + 8 <= N_CLS)
            def _():
                pltpu.sync_copy(
                    cb_v.at[pl.ds(j * 8, 8)],
                    out_hbm.at[1 + j].at[pl.ds(base + off8, 8)],
                )

        gather(0, 0).start()

        def body(jj, carry):
            for b in (0, 1):
                i = 2 * jj + b
                gather(i + 1, 1 - b).start()
                gather(i, b).wait()
                write_gather(i, b)
                write_ctx_chunk(i)
            return carry

        lax.fori_loop(0, (NGPOS - 1) // 2, body, 0)
        gather(NGPOS - 1, 0).wait()
        write_gather(NGPOS - 1, 0)
        for q in range(NGPOS - 1, NCTX_CHUNKS):
            write_ctx_chunk(q)

    return sc_kernel


_sc_call = _make_sc_call()


def kernel(tokenized_prompts, token_embedding, ctx):
    tok = tokenized_prompts.astype(jnp.int32)
    tokp = jnp.concatenate([tok, tok[: NW * CPW - N_CLS]], axis=0)
    gidx = tokp.reshape(NW, CPW, SEQ).transpose(0, 2, 1)
    gidx = gidx.reshape(NW, SEQ, 1, CPW)
    cbig = jnp.repeat(ctx, 8, axis=0)
    out77 = _sc_call(gidx, token_embedding, cbig)
    return out77.transpose(1, 0, 2)
